# baseline (speedup 1.0000x reference)
import jax
import jax.numpy as jnp
from jax import lax
from jax.experimental import pallas as pl
from jax.experimental.pallas import tpu as pltpu
from jax.experimental.pallas import tpu_sc as plsc

_N, _E, _G = 2048, 32768, 8
_NW = 32
_F32 = jnp.float32


def _dot(a, b):
    return jnp.dot(a, b, preferred_element_type=_F32)


def _ln(h, s, b):
    mu = jnp.mean(h, axis=-1, keepdims=True)
    d = h - mu
    var = jnp.mean(d * d, axis=-1, keepdims=True)
    return d * lax.rsqrt(var + 1e-5) * s + b


def _onehot(idx, rows):
    return (idx[:, None] == lax.broadcasted_iota(jnp.int32, (rows, _G), 1)).astype(_F32)


def _pack_bf16(x):
    def rnd(v):
        u = lax.bitcast_convert_type(v, jnp.uint32)
        return (u + jnp.uint32(0x7FFF) + ((u >> 16) & jnp.uint32(1))) >> 16

    w = rnd(x[:, :512]) | (rnd(x[:, 512:]) << 16)
    return lax.bitcast_convert_type(w, jnp.int32)


def _unpack_bf16(w):
    u = lax.bitcast_convert_type(w, jnp.uint32)
    lo = lax.bitcast_convert_type(u << 16, _F32)
    hi = lax.bitcast_convert_type((u >> 16) << 16, _F32)
    return jnp.concatenate([lo, hi], axis=1)


def _glob_pre_body(g_ref, w0, b0, w1, b1, s, t, d_ref, vc_ref, ua_ref, w2_ref,
                   gd_o, gv_o, gu_o, w2b_o):
    x = g_ref[...]
    h = _dot(x, w0[...]) + b0[...]
    h = _dot(jax.nn.relu(h), w1[...]) + b1[...]
    h = _ln(h, s[...], t[...])
    gc = jnp.concatenate([x, h], axis=1)
    gd_o[...] = _dot(gc, d_ref[...])
    gv_o[...] = _dot(gc, vc_ref[...])
    gu_o[...] = _dot(gc, ua_ref[...])
    w2b_o[...] = w2_ref[...].astype(jnp.bfloat16)


def _glob_pre(gf, w0, b0, w1, b1, s, t, W1e, V1, U1, W2e):
    fb2 = lambda shp, ij: pl.BlockSpec(shp, lambda i, _ij=ij: _ij)
    fb1 = lambda shp: pl.BlockSpec(shp, lambda i: (0,))
    return pl.pallas_call(
        _glob_pre_body,
        grid=(1,),
        in_specs=[fb2((_G, 128), (0, 0)),
                  fb2((128, 128), (0, 0)), fb1((128,)),
                  fb2((128, 128), (0, 0)), fb1((128,)),
                  fb1((128,)), fb1((128,)),
                  fb2((256, 1024), (9, 0)),
                  fb2((256, 1024), (6, 0)),
                  fb2((256, 512), (0, 0)),
                  fb2((1024, 512), (0, 0))],
        out_specs=[fb2((_G, 1024), (0, 0)), fb2((_G, 1024), (0, 0)),
                   fb2((_G, 512), (0, 0)), fb2((1024, 512), (0, 0))],
        out_shape=[jax.ShapeDtypeStruct((_G, 1024), _F32),
                   jax.ShapeDtypeStruct((_G, 1024), _F32),
                   jax.ShapeDtypeStruct((_G, 512), _F32),
                   jax.ShapeDtypeStruct((1024, 512), jnp.bfloat16)],
    )(gf, w0, b0, w1, b1, s, t, W1e, V1, U1, W2e)


def _node_pre_body(x_ref, w0, b0, w1, b1, s, t, a_ref, bm_ref, va_ref,
                   nr_o, ns_o, nv_o):
    x = x_ref[...]
    h = _dot(x, w0[...]) + b0[...]
    h = _dot(jax.nn.relu(h), w1[...]) + b1[...]
    h = _ln(h, s[...], t[...])
    nb = jnp.concatenate([x, h], axis=1)
    nr_o[...] = _pack_bf16(_dot(nb, a_ref[...]))
    ns_o[...] = _pack_bf16(_dot(nb, bm_ref[...]))
    nv_o[...] = _dot(nb, va_ref[...])


def _node_pre(nodes, w0, b0, w1, b1, s, t, A, Bm, V1a):
    blk = 256
    fx2 = lambda shp: pl.BlockSpec(shp, lambda i: (0, 0))
    fx1 = lambda shp: pl.BlockSpec(shp, lambda i: (0,))
    return pl.pallas_call(
        _node_pre_body,
        grid=(_N // blk,),
        in_specs=[pl.BlockSpec((blk, 512), lambda i: (i, 0)),
                  fx2((512, 512)), fx1((512,)), fx2((512, 512)), fx1((512,)),
                  fx1((512,)), fx1((512,)),
                  pl.BlockSpec((1024, 1024), lambda i: (0, 0)),
                  pl.BlockSpec((1024, 1024), lambda i: (1, 0)),
                  pl.BlockSpec((1024, 1024), lambda i: (0, 0))],
        out_specs=[pl.BlockSpec((blk, 512), lambda i: (i, 0)),
                   pl.BlockSpec((blk, 512), lambda i: (i, 0)),
                   pl.BlockSpec((blk, 1024), lambda i: (i, 0))],
        out_shape=[jax.ShapeDtypeStruct((_N, 512), jnp.int32),
                   jax.ShapeDtypeStruct((_N, 512), jnp.int32),
                   jax.ShapeDtypeStruct((_N, 1024), _F32)],
    )(nodes, w0, b0, w1, b1, s, t, A, Bm, V1a)


def _edge_pre_body(x_ref, w0, b0, w1, b1, s, t, c_ref, gd_ref, b1e_ref, idx_ref,
                   ee_o):
    x = x_ref[...]
    h = _dot(x, w0[...]) + b0[...]
    h = _dot(jax.nn.relu(h), w1[...]) + b1[...]
    h = _ln(h, s[...], t[...])
    eb = jnp.concatenate([x, h], axis=1)
    oh = _onehot(idx_ref[0, 0, :], eb.shape[0])
    ee_o[...] = _pack_bf16(_dot(eb, c_ref[...]) + _dot(oh, gd_ref[...])
                           + b1e_ref[...])


def _edge_pre(edges, w0, b0, w1, b1, s, t, C, gd, b1e, egidx3):
    blk = 1024
    fx2 = lambda shp: pl.BlockSpec(shp, lambda i: (0, 0))
    fx1 = lambda shp: pl.BlockSpec(shp, lambda i: (0,))
    return pl.pallas_call(
        _edge_pre_body,
        grid=(_E // blk,),
        in_specs=[pl.BlockSpec((blk, 128), lambda i: (i, 0)),
                  fx2((128, 128)), fx1((128,)), fx2((128, 128)), fx1((128,)),
                  fx1((128,)), fx1((128,)),
                  pl.BlockSpec((256, 1024), lambda i: (8, 0)),
                  fx2((_G, 1024)), fx1((1024,)),
                  pl.BlockSpec((1, 1, blk), lambda i: (i, 0, 0))],
        out_specs=pl.BlockSpec((blk, 512), lambda i: (i, 0)),
        out_shape=jax.ShapeDtypeStruct((_E, 512), jnp.int32),
    )(edges, w0, b0, w1, b1, s, t, C, gd, b1e, egidx3)


def _edge_tail_body(ee_ref, g1_ref, g2_ref, w2, b2, s, t, idx_ref, recv_ref,
                    enew_o, eagg_o, agg_o):
    h = (_unpack_bf16(ee_ref[...]) + _unpack_bf16(g1_ref[...])
         + _unpack_bf16(g2_ref[...]))
    y = _dot(jax.nn.relu(h).astype(jnp.bfloat16), w2[...]) + b2[...]
    y = _ln(y, s[...], t[...])
    enew_o[...] = y
    oh = _onehot(idx_ref[0, 0, :], y.shape[0])
    part = lax.dot_general(oh, y, (((0,), (0,)), ((), ())),
                           preferred_element_type=_F32)

    rids = recv_ref[0, 0, :]
    ohr = (rids[:, None] == lax.broadcasted_iota(jnp.int32, (rids.shape[0], _N), 1)
           ).astype(jnp.bfloat16)
    aggpart = lax.dot_general(ohr, y.astype(jnp.bfloat16),
                              (((0,), (0,)), ((), ())),
                              preferred_element_type=_F32)

    @pl.when(pl.program_id(0) == 0)
    def _():
        eagg_o[...] = jnp.zeros(eagg_o.shape, eagg_o.dtype)
        agg_o[...] = jnp.zeros(agg_o.shape, agg_o.dtype)

    eagg_o[...] += part
    agg_o[...] += aggpart


def _edge_tail(ee, g1, g2, W2e, b2e, s, t, egidx3, recv3):
    blk = 2048
    fx2 = lambda shp: pl.BlockSpec(shp, lambda i: (0, 0))
    fx1 = lambda shp: pl.BlockSpec(shp, lambda i: (0,))
    return pl.pallas_call(
        _edge_tail_body,
        grid=(_E // blk,),
        in_specs=[pl.BlockSpec((blk, 512), lambda i: (i, 0)),
                  pl.BlockSpec((blk, 512), lambda i: (i, 0)),
                  pl.BlockSpec((blk, 512), lambda i: (i, 0)),
                  fx2((1024, 512)), fx1((512,)), fx1((512,)), fx1((512,)),
                  pl.BlockSpec((1, 1, blk), lambda i: (i, 0, 0)),
                  pl.BlockSpec((1, 1, blk), lambda i: (i, 0, 0))],
        out_specs=[pl.BlockSpec((blk, 512), lambda i: (i, 0)),
                   pl.BlockSpec((_G, 512), lambda i: (0, 0)),
                   pl.BlockSpec((_N, 512), lambda i: (0, 0))],
        out_shape=[jax.ShapeDtypeStruct((_E, 512), _F32),
                   jax.ShapeDtypeStruct((_G, 512), _F32),
                   jax.ShapeDtypeStruct((_N, 512), _F32)],
    )(ee, g1, g2, W2e, b2e, s, t, egidx3, recv3)


def _node_tail_body(nv_ref, a0_ref, vb, gv_ref, idx_ref, c1, v2, c2, s, t,
                    gu_ref, ea_ref, u1_ref, d1, u2, d2, su, tu,
                    nnew_o, nagg_o, gnew_o):
    agg = a0_ref[...]
    oh = _onehot(idx_ref[0, 0, :], agg.shape[0])
    h = nv_ref[...] + _dot(agg, vb[...]) + _dot(oh, gv_ref[...]) + c1[...]
    y = _dot(jax.nn.relu(h), v2[...]) + c2[...]
    y = _ln(y, s[...], t[...])
    nnew_o[...] = y
    part = lax.dot_general(oh, y, (((0,), (0,)), ((), ())),
                           preferred_element_type=_F32)

    @pl.when(pl.program_id(0) == 0)
    def _():
        nagg_o[...] = jnp.zeros(nagg_o.shape, nagg_o.dtype)

    nagg_o[...] += part

    @pl.when(pl.program_id(0) == pl.num_programs(0) - 1)
    def _():
        ub = u1_ref[pl.ds(256, 512), :]
        uc = u1_ref[pl.ds(768, 512), :]
        hg = (gu_ref[...] + _dot(nagg_o[...], ub) + _dot(ea_ref[...], uc)
              + d1[...])
        yg = _dot(jax.nn.relu(hg), u2[...]) + d2[...]
        gnew_o[...] = _ln(yg, su[...], tu[...])


def _node_tail(nv, aggp, V1, gv, ngidx3, c1, V2, c2, s, t,
               gu, eagg, U1, d1, U2, d2, su, tu):
    blk = 256
    fx2 = lambda shp: pl.BlockSpec(shp, lambda i: (0, 0))
    fx1 = lambda shp: pl.BlockSpec(shp, lambda i: (0,))
    nblk = _N // blk
    return pl.pallas_call(
        _node_tail_body,
        grid=(nblk,),
        in_specs=[pl.BlockSpec((blk, 1024), lambda i: (i, 0)),
                  pl.BlockSpec((blk, 512), lambda i: (i, 0)),
                  pl.BlockSpec((512, 1024), lambda i: (2, 0)),
                  fx2((_G, 1024)),
                  pl.BlockSpec((1, 1, blk), lambda i: (i, 0, 0)),
                  fx1((1024,)), fx2((1024, 512)), fx1((512,)),
                  fx1((512,)), fx1((512,)),
                  fx2((_G, 512)), fx2((_G, 512)), fx2((1280, 512)),
                  fx1((512,)), fx2((512, 256)), fx1((256,)),
                  fx1((256,)), fx1((256,))],
        out_specs=[pl.BlockSpec((blk, 512), lambda i: (i, 0)),
                   pl.BlockSpec((_G, 512), lambda i: (0, 0)),
                   pl.BlockSpec((_G, 256), lambda i: (0, 0))],
        out_shape=[jax.ShapeDtypeStruct((_N, 512), _F32),
                   jax.ShapeDtypeStruct((_G, 512), _F32),
                   jax.ShapeDtypeStruct((_G, 256), _F32)],
    )(nv, aggp, V1, gv, ngidx3, c1, V2, c2, s, t,
      gu, eagg, U1, d1, U2, d2, su, tu)


_GCH = 64
_GPER = _E // _NW


def _sc_gather_body(nr_hbm, ns_hbm, recv_hbm, send_hbm, g1_hbm, g2_hbm,
                    idxr_v, idxs_v, buf0, buf1, buf2,
                    gsem0, gsem1, gsem2, wsem0, wsem1, wsem2):
    wid = lax.axis_index("s") * 2 + lax.axis_index("c")
    base = wid * _GPER
    pltpu.sync_copy(recv_hbm.at[pl.ds(base, _GPER)], idxr_v)
    pltpu.sync_copy(send_hbm.at[pl.ds(base, _GPER)], idxs_v)

    nchunks = _GPER // _GCH
    bufs = (buf0, buf1, buf2)
    gsems = (gsem0, gsem1, gsem2)
    wsems = (wsem0, wsem1, wsem2)

    def plan(k):
        if k < nchunks:
            return nr_hbm, idxr_v, g1_hbm, k * _GCH
        return ns_hbm, idxs_v, g2_hbm, (k - nchunks) * _GCH

    def start_gather(k):
        src, idx, _, off = plan(k)
        return pltpu.async_copy(src.at[idx.at[pl.ds(off, _GCH)]],
                                bufs[k % 3], gsems[k % 3])

    def start_write(k):
        _, _, dst, off = plan(k)
        return pltpu.async_copy(bufs[k % 3], dst.at[pl.ds(base + off, _GCH)],
                                wsems[k % 3])

    total = 2 * nchunks
    gat = [start_gather(0), start_gather(1)]
    wr = []
    for k in range(total):
        gat[k].wait()
        if k >= 1:
            wr[k - 1].wait()
        if k + 2 < total:
            gat.append(start_gather(k + 2))
        else:
            gat.append(None)
        wr.append(start_write(k))
    wr[-1].wait()


def _sc_gather(nri, nsi, recv, send):
    mesh = plsc.VectorSubcoreMesh(core_axis_name="c", subcore_axis_name="s")
    f = pl.kernel(
        _sc_gather_body,
        out_type=[jax.ShapeDtypeStruct((_E, 512), jnp.int32)] * 2,
        mesh=mesh,
        scratch_types=[pltpu.VMEM((_GPER,), jnp.int32),
                       pltpu.VMEM((_GPER,), jnp.int32),
                       pltpu.VMEM((_GCH, 512), jnp.int32),
                       pltpu.VMEM((_GCH, 512), jnp.int32),
                       pltpu.VMEM((_GCH, 512), jnp.int32),
                       pltpu.SemaphoreType.DMA, pltpu.SemaphoreType.DMA,
                       pltpu.SemaphoreType.DMA, pltpu.SemaphoreType.DMA,
                       pltpu.SemaphoreType.DMA, pltpu.SemaphoreType.DMA],
    )
    return f(nri, nsi, recv, send)


def kernel(nodes, edges, globals_feat, params, receivers, senders,
           node_graph_idx, edge_graph_idx):
    pn = params["node_encoder"]
    (wn0, bn0), (wn1, bn1) = pn["lin"]
    sn, tn = pn["ln"]
    pe = params["edge_encoder"]
    (we0, be0), (we1, be1) = pe["lin"]
    se, te = pe["ln"]
    pg = params["global_encoder"]
    (wg0, bg0), (wg1, bg1) = pg["lin"]
    sg, tg = pg["ln"]
    pm = params["edge_model"]
    (W1e, b1e), (W2e, b2e) = pm["lin"]
    sm, tm = pm["ln"]
    pv = params["node_model"]
    (V1, c1), (V2, c2) = pv["lin"]
    sv, tv = pv["ln"]
    pu = params["global_model"]
    (U1, d1), (U2, d2) = pu["lin"]
    su, tu = pu["ln"]

    recv = receivers.astype(jnp.int32)
    send = senders.astype(jnp.int32)
    recv3 = recv.reshape(_E // 2048, 1, 2048)
    egidx3b = edge_graph_idx.astype(jnp.int32).reshape(_E // 1024, 1, 1024)
    egidx3c = edge_graph_idx.astype(jnp.int32).reshape(_E // 2048, 1, 2048)
    ngidx3 = node_graph_idx.astype(jnp.int32).reshape(_N // 256, 1, 256)

    gd, gv, gu, W2eb = _glob_pre(globals_feat, wg0, bg0, wg1, bg1, sg, tg,
                                 W1e, V1, U1, W2e)
    nr, ns, nv = _node_pre(nodes, wn0, bn0, wn1, bn1, sn, tn, W1e, W1e, V1)
    g1, g2 = _sc_gather(nr, ns, recv, send)
    ee = _edge_pre(edges, we0, be0, we1, be1, se, te, W1e, gd, b1e, egidx3b)
    e_new, eagg, agg = _edge_tail(ee, g1, g2,
                                  W2eb, b2e, sm, tm, egidx3c, recv3)
    n_new, nagg, g_new = _node_tail(nv, agg, V1, gv, ngidx3, c1, V2, c2,
                                    sv, tv, gu, eagg, U1, d1, U2, d2, su, tu)
    return (n_new, e_new, g_new)


# speedup vs baseline: 1.0038x; 1.0038x over previous
import jax
import jax.numpy as jnp
from jax import lax
from jax.experimental import pallas as pl
from jax.experimental.pallas import tpu as pltpu
from jax.experimental.pallas import tpu_sc as plsc

_N, _E, _G = 2048, 32768, 8
_NW = 32
_F32 = jnp.float32


def _dot(a, b):
    return jnp.dot(a, b, preferred_element_type=_F32)


def _ln(h, s, b):
    mu = jnp.mean(h, axis=-1, keepdims=True)
    d = h - mu
    var = jnp.mean(d * d, axis=-1, keepdims=True)
    return d * lax.rsqrt(var + 1e-5) * s + b


def _onehot(idx, rows):
    return (idx[:, None] == lax.broadcasted_iota(jnp.int32, (rows, _G), 1)).astype(_F32)


def _pack_bf16(x):
    def rnd(v):
        u = lax.bitcast_convert_type(v, jnp.uint32)
        return (u + jnp.uint32(0x7FFF) + ((u >> 16) & jnp.uint32(1))) >> 16

    w = rnd(x[:, :512]) | (rnd(x[:, 512:]) << 16)
    return lax.bitcast_convert_type(w, jnp.int32)


def _unpack_bf16(w):
    u = lax.bitcast_convert_type(w, jnp.uint32)
    lo = lax.bitcast_convert_type(u << 16, _F32)
    hi = lax.bitcast_convert_type((u >> 16) << 16, _F32)
    return jnp.concatenate([lo, hi], axis=1)


def _glob_pre_body(g_ref, w0, b0, w1, b1, s, t, d_ref, vc_ref, ua_ref, w2_ref,
                   gd_o, gv_o, gu_o, w2b_o):
    x = g_ref[...]
    h = _dot(x, w0[...]) + b0[...]
    h = _dot(jax.nn.relu(h), w1[...]) + b1[...]
    h = _ln(h, s[...], t[...])
    gc = jnp.concatenate([x, h], axis=1)
    gd_o[...] = _dot(gc, d_ref[...])
    gv_o[...] = _dot(gc, vc_ref[...])
    gu_o[...] = _dot(gc, ua_ref[...])
    w2b_o[...] = w2_ref[...].astype(jnp.bfloat16)


def _glob_pre(gf, w0, b0, w1, b1, s, t, W1e, V1, U1, W2e):
    fb2 = lambda shp, ij: pl.BlockSpec(shp, lambda i, _ij=ij: _ij)
    fb1 = lambda shp: pl.BlockSpec(shp, lambda i: (0,))
    return pl.pallas_call(
        _glob_pre_body,
        grid=(1,),
        in_specs=[fb2((_G, 128), (0, 0)),
                  fb2((128, 128), (0, 0)), fb1((128,)),
                  fb2((128, 128), (0, 0)), fb1((128,)),
                  fb1((128,)), fb1((128,)),
                  fb2((256, 1024), (9, 0)),
                  fb2((256, 1024), (6, 0)),
                  fb2((256, 512), (0, 0)),
                  fb2((1024, 512), (0, 0))],
        out_specs=[fb2((_G, 1024), (0, 0)), fb2((_G, 1024), (0, 0)),
                   fb2((_G, 512), (0, 0)), fb2((1024, 512), (0, 0))],
        out_shape=[jax.ShapeDtypeStruct((_G, 1024), _F32),
                   jax.ShapeDtypeStruct((_G, 1024), _F32),
                   jax.ShapeDtypeStruct((_G, 512), _F32),
                   jax.ShapeDtypeStruct((1024, 512), jnp.bfloat16)],
    )(gf, w0, b0, w1, b1, s, t, W1e, V1, U1, W2e)


def _node_pre_body(x_ref, w0, b0, w1, b1, s, t, a_ref, bm_ref, va_ref,
                   nr_o, ns_o, nv_o):
    x = x_ref[...]
    h = _dot(x, w0[...]) + b0[...]
    h = _dot(jax.nn.relu(h), w1[...]) + b1[...]
    h = _ln(h, s[...], t[...])
    nb = jnp.concatenate([x, h], axis=1)
    nr_o[...] = _pack_bf16(_dot(nb, a_ref[...]))
    ns_o[...] = _pack_bf16(_dot(nb, bm_ref[...]))
    nv_o[...] = _dot(nb, va_ref[...])


def _node_pre(nodes, w0, b0, w1, b1, s, t, A, Bm, V1a):
    blk = 256
    fx2 = lambda shp: pl.BlockSpec(shp, lambda i: (0, 0))
    fx1 = lambda shp: pl.BlockSpec(shp, lambda i: (0,))
    return pl.pallas_call(
        _node_pre_body,
        grid=(_N // blk,),
        in_specs=[pl.BlockSpec((blk, 512), lambda i: (i, 0)),
                  fx2((512, 512)), fx1((512,)), fx2((512, 512)), fx1((512,)),
                  fx1((512,)), fx1((512,)),
                  pl.BlockSpec((1024, 1024), lambda i: (0, 0)),
                  pl.BlockSpec((1024, 1024), lambda i: (1, 0)),
                  pl.BlockSpec((1024, 1024), lambda i: (0, 0))],
        out_specs=[pl.BlockSpec((blk, 512), lambda i: (i, 0)),
                   pl.BlockSpec((blk, 512), lambda i: (i, 0)),
                   pl.BlockSpec((blk, 1024), lambda i: (i, 0))],
        out_shape=[jax.ShapeDtypeStruct((_N, 512), jnp.int32),
                   jax.ShapeDtypeStruct((_N, 512), jnp.int32),
                   jax.ShapeDtypeStruct((_N, 1024), _F32)],
    )(nodes, w0, b0, w1, b1, s, t, A, Bm, V1a)


def _edge_pre_body(x_ref, w0, b0, w1, b1, s, t, c_ref, gd_ref, b1e_ref, idx_ref,
                   ee_o):
    x = x_ref[...]
    h = _dot(x, w0[...]) + b0[...]
    h = _dot(jax.nn.relu(h), w1[...]) + b1[...]
    h = _ln(h, s[...], t[...])
    eb = jnp.concatenate([x, h], axis=1)
    oh = _onehot(idx_ref[0, 0, :], eb.shape[0])
    ee_o[...] = _pack_bf16(_dot(eb, c_ref[...]) + _dot(oh, gd_ref[...])
                           + b1e_ref[...])


def _edge_pre(edges, w0, b0, w1, b1, s, t, C, gd, b1e, egidx3):
    blk = 1024
    fx2 = lambda shp: pl.BlockSpec(shp, lambda i: (0, 0))
    fx1 = lambda shp: pl.BlockSpec(shp, lambda i: (0,))
    return pl.pallas_call(
        _edge_pre_body,
        grid=(_E // blk,),
        in_specs=[pl.BlockSpec((blk, 128), lambda i: (i, 0)),
                  fx2((128, 128)), fx1((128,)), fx2((128, 128)), fx1((128,)),
                  fx1((128,)), fx1((128,)),
                  pl.BlockSpec((256, 1024), lambda i: (8, 0)),
                  fx2((_G, 1024)), fx1((1024,)),
                  pl.BlockSpec((1, 1, blk), lambda i: (i, 0, 0))],
        out_specs=pl.BlockSpec((blk, 512), lambda i: (i, 0)),
        out_shape=jax.ShapeDtypeStruct((_E, 512), jnp.int32),
    )(edges, w0, b0, w1, b1, s, t, C, gd, b1e, egidx3)


def _edge_tail_body(ee_ref, g1_ref, g2_ref, w2, b2, s, t, idx_ref, recv_ref,
                    enew_o, eagg_o, agg_o):
    h = (_unpack_bf16(ee_ref[...]) + _unpack_bf16(g1_ref[...])
         + _unpack_bf16(g2_ref[...]))
    y = _dot(jax.nn.relu(h).astype(jnp.bfloat16), w2[...]) + b2[...]
    y = _ln(y, s[...], t[...])
    enew_o[...] = y
    oh = _onehot(idx_ref[0, 0, :], y.shape[0])
    part = lax.dot_general(oh, y, (((0,), (0,)), ((), ())),
                           preferred_element_type=_F32)

    rids = recv_ref[0, 0, :]
    ohr = (rids[:, None] == lax.broadcasted_iota(jnp.int32, (rids.shape[0], _N), 1)
           ).astype(jnp.bfloat16)
    aggpart = lax.dot_general(ohr, y.astype(jnp.bfloat16),
                              (((0,), (0,)), ((), ())),
                              preferred_element_type=_F32)

    @pl.when(pl.program_id(0) == 0)
    def _():
        eagg_o[...] = jnp.zeros(eagg_o.shape, eagg_o.dtype)
        agg_o[...] = jnp.zeros(agg_o.shape, agg_o.dtype)

    eagg_o[...] += part
    agg_o[...] += aggpart


def _edge_tail(ee, g1, g2, W2e, b2e, s, t, egidx3, recv3):
    blk = 1024
    fx2 = lambda shp: pl.BlockSpec(shp, lambda i: (0, 0))
    fx1 = lambda shp: pl.BlockSpec(shp, lambda i: (0,))
    return pl.pallas_call(
        _edge_tail_body,
        grid=(_E // blk,),
        in_specs=[pl.BlockSpec((blk, 512), lambda i: (i, 0)),
                  pl.BlockSpec((blk, 512), lambda i: (i, 0)),
                  pl.BlockSpec((blk, 512), lambda i: (i, 0)),
                  fx2((1024, 512)), fx1((512,)), fx1((512,)), fx1((512,)),
                  pl.BlockSpec((1, 1, blk), lambda i: (i, 0, 0)),
                  pl.BlockSpec((1, 1, blk), lambda i: (i, 0, 0))],
        out_specs=[pl.BlockSpec((blk, 512), lambda i: (i, 0)),
                   pl.BlockSpec((_G, 512), lambda i: (0, 0)),
                   pl.BlockSpec((_N, 512), lambda i: (0, 0))],
        out_shape=[jax.ShapeDtypeStruct((_E, 512), _F32),
                   jax.ShapeDtypeStruct((_G, 512), _F32),
                   jax.ShapeDtypeStruct((_N, 512), _F32)],
    )(ee, g1, g2, W2e, b2e, s, t, egidx3, recv3)


def _node_tail_body(nv_ref, a0_ref, vb, gv_ref, idx_ref, c1, v2, c2, s, t,
                    gu_ref, ea_ref, u1_ref, d1, u2, d2, su, tu,
                    nnew_o, nagg_o, gnew_o):
    agg = a0_ref[...]
    oh = _onehot(idx_ref[0, 0, :], agg.shape[0])
    h = nv_ref[...] + _dot(agg, vb[...]) + _dot(oh, gv_ref[...]) + c1[...]
    y = _dot(jax.nn.relu(h), v2[...]) + c2[...]
    y = _ln(y, s[...], t[...])
    nnew_o[...] = y
    part = lax.dot_general(oh, y, (((0,), (0,)), ((), ())),
                           preferred_element_type=_F32)

    @pl.when(pl.program_id(0) == 0)
    def _():
        nagg_o[...] = jnp.zeros(nagg_o.shape, nagg_o.dtype)

    nagg_o[...] += part

    @pl.when(pl.program_id(0) == pl.num_programs(0) - 1)
    def _():
        ub = u1_ref[pl.ds(256, 512), :]
        uc = u1_ref[pl.ds(768, 512), :]
        hg = (gu_ref[...] + _dot(nagg_o[...], ub) + _dot(ea_ref[...], uc)
              + d1[...])
        yg = _dot(jax.nn.relu(hg), u2[...]) + d2[...]
        gnew_o[...] = _ln(yg, su[...], tu[...])


def _node_tail(nv, aggp, V1, gv, ngidx3, c1, V2, c2, s, t,
               gu, eagg, U1, d1, U2, d2, su, tu):
    blk = 256
    fx2 = lambda shp: pl.BlockSpec(shp, lambda i: (0, 0))
    fx1 = lambda shp: pl.BlockSpec(shp, lambda i: (0,))
    nblk = _N // blk
    return pl.pallas_call(
        _node_tail_body,
        grid=(nblk,),
        in_specs=[pl.BlockSpec((blk, 1024), lambda i: (i, 0)),
                  pl.BlockSpec((blk, 512), lambda i: (i, 0)),
                  pl.BlockSpec((512, 1024), lambda i: (2, 0)),
                  fx2((_G, 1024)),
                  pl.BlockSpec((1, 1, blk), lambda i: (i, 0, 0)),
                  fx1((1024,)), fx2((1024, 512)), fx1((512,)),
                  fx1((512,)), fx1((512,)),
                  fx2((_G, 512)), fx2((_G, 512)), fx2((1280, 512)),
                  fx1((512,)), fx2((512, 256)), fx1((256,)),
                  fx1((256,)), fx1((256,))],
        out_specs=[pl.BlockSpec((blk, 512), lambda i: (i, 0)),
                   pl.BlockSpec((_G, 512), lambda i: (0, 0)),
                   pl.BlockSpec((_G, 256), lambda i: (0, 0))],
        out_shape=[jax.ShapeDtypeStruct((_N, 512), _F32),
                   jax.ShapeDtypeStruct((_G, 512), _F32),
                   jax.ShapeDtypeStruct((_G, 256), _F32)],
    )(nv, aggp, V1, gv, ngidx3, c1, V2, c2, s, t,
      gu, eagg, U1, d1, U2, d2, su, tu)


_GCH = 64
_GPER = _E // _NW


def _sc_gather_body(nr_hbm, ns_hbm, recv_hbm, send_hbm, g1_hbm, g2_hbm,
                    idxr_v, idxs_v, buf0, buf1, buf2,
                    gsem0, gsem1, gsem2, wsem0, wsem1, wsem2):
    wid = lax.axis_index("s") * 2 + lax.axis_index("c")
    base = wid * _GPER
    pltpu.sync_copy(recv_hbm.at[pl.ds(base, _GPER)], idxr_v)
    pltpu.sync_copy(send_hbm.at[pl.ds(base, _GPER)], idxs_v)

    nchunks = _GPER // _GCH
    bufs = (buf0, buf1, buf2)
    gsems = (gsem0, gsem1, gsem2)
    wsems = (wsem0, wsem1, wsem2)

    def plan(k):
        if k < nchunks:
            return nr_hbm, idxr_v, g1_hbm, k * _GCH
        return ns_hbm, idxs_v, g2_hbm, (k - nchunks) * _GCH

    def start_gather(k):
        src, idx, _, off = plan(k)
        return pltpu.async_copy(src.at[idx.at[pl.ds(off, _GCH)]],
                                bufs[k % 3], gsems[k % 3])

    def start_write(k):
        _, _, dst, off = plan(k)
        return pltpu.async_copy(bufs[k % 3], dst.at[pl.ds(base + off, _GCH)],
                                wsems[k % 3])

    total = 2 * nchunks
    gat = [start_gather(0), start_gather(1)]
    wr = []
    for k in range(total):
        gat[k].wait()
        if k >= 1:
            wr[k - 1].wait()
        if k + 2 < total:
            gat.append(start_gather(k + 2))
        else:
            gat.append(None)
        wr.append(start_write(k))
    wr[-1].wait()


def _sc_gather(nri, nsi, recv, send):
    mesh = plsc.VectorSubcoreMesh(core_axis_name="c", subcore_axis_name="s")
    f = pl.kernel(
        _sc_gather_body,
        out_type=[jax.ShapeDtypeStruct((_E, 512), jnp.int32)] * 2,
        mesh=mesh,
        scratch_types=[pltpu.VMEM((_GPER,), jnp.int32),
                       pltpu.VMEM((_GPER,), jnp.int32),
                       pltpu.VMEM((_GCH, 512), jnp.int32),
                       pltpu.VMEM((_GCH, 512), jnp.int32),
                       pltpu.VMEM((_GCH, 512), jnp.int32),
                       pltpu.SemaphoreType.DMA, pltpu.SemaphoreType.DMA,
                       pltpu.SemaphoreType.DMA, pltpu.SemaphoreType.DMA,
                       pltpu.SemaphoreType.DMA, pltpu.SemaphoreType.DMA],
    )
    return f(nri, nsi, recv, send)


def kernel(nodes, edges, globals_feat, params, receivers, senders,
           node_graph_idx, edge_graph_idx):
    pn = params["node_encoder"]
    (wn0, bn0), (wn1, bn1) = pn["lin"]
    sn, tn = pn["ln"]
    pe = params["edge_encoder"]
    (we0, be0), (we1, be1) = pe["lin"]
    se, te = pe["ln"]
    pg = params["global_encoder"]
    (wg0, bg0), (wg1, bg1) = pg["lin"]
    sg, tg = pg["ln"]
    pm = params["edge_model"]
    (W1e, b1e), (W2e, b2e) = pm["lin"]
    sm, tm = pm["ln"]
    pv = params["node_model"]
    (V1, c1), (V2, c2) = pv["lin"]
    sv, tv = pv["ln"]
    pu = params["global_model"]
    (U1, d1), (U2, d2) = pu["lin"]
    su, tu = pu["ln"]

    recv = receivers.astype(jnp.int32)
    send = senders.astype(jnp.int32)
    recv3 = recv.reshape(_E // 1024, 1, 1024)
    egidx3b = edge_graph_idx.astype(jnp.int32).reshape(_E // 1024, 1, 1024)
    ngidx3 = node_graph_idx.astype(jnp.int32).reshape(_N // 256, 1, 256)

    gd, gv, gu, W2eb = _glob_pre(globals_feat, wg0, bg0, wg1, bg1, sg, tg,
                                 W1e, V1, U1, W2e)
    nr, ns, nv = _node_pre(nodes, wn0, bn0, wn1, bn1, sn, tn, W1e, W1e, V1)
    g1, g2 = _sc_gather(nr, ns, recv, send)
    ee = _edge_pre(edges, we0, be0, we1, be1, se, te, W1e, gd, b1e, egidx3b)
    e_new, eagg, agg = _edge_tail(ee, g1, g2,
                                  W2eb, b2e, sm, tm, egidx3b, recv3)
    n_new, nagg, g_new = _node_tail(nv, agg, V1, gv, ngidx3, c1, V2, c2,
                                    sv, tv, gu, eagg, U1, d1, U2, d2, su, tu)
    return (n_new, e_new, g_new)
